# baseline (speedup 1.0000x reference)
import functools

import jax
import jax.numpy as jnp
from jax import lax
from jax.experimental import pallas as pl
from jax.experimental.pallas import tpu as pltpu
from jax.experimental.pallas import tpu_sc as plsc

_TOPK = 16
_NEG = -1e30


def _prep_body(ltm_ref, mn_ref):
    v = ltm_ref[...]
    inv = jax.lax.rsqrt(jnp.maximum(jnp.sum(v * v, axis=1, keepdims=True), 1e-24))
    mn_ref[...] = (v * inv).astype(jnp.bfloat16)


def _main_body(x_ref, mn_hbm, w1_hbm, w2_hbm, b1_ref, b2_ref,
               mem_ref, vals_ref, idx_ref, mn_v, w1_v, w2_v, sem):
    @pl.when(pl.program_id(0) == 0)
    def _stage():
        cps = [
            pltpu.make_async_copy(mn_hbm, mn_v, sem.at[0]),
            pltpu.make_async_copy(w1_hbm, w1_v, sem.at[1]),
            pltpu.make_async_copy(w2_hbm, w2_v, sem.at[2]),
        ]
        for c in cps:
            c.start()
        for c in cps:
            c.wait()

    x = x_ref[...]
    tq = x.shape[0]
    m = mn_v.shape[0]

    h = jnp.dot(x.astype(jnp.bfloat16), w1_v[...],
                preferred_element_type=jnp.float32) + b1_ref[...]
    h = h * 0.5 * (1.0 + jax.lax.erf(h * 0.7071067811865476))
    mem = jnp.dot(h.astype(jnp.bfloat16), w2_v[...],
                  preferred_element_type=jnp.float32) + b2_ref[...]
    mem_ref[...] = mem

    qn = x * jax.lax.rsqrt(jnp.maximum(jnp.sum(x * x, axis=1, keepdims=True), 1e-24))
    s = jax.lax.dot_general(qn.astype(jnp.bfloat16), mn_v[...],
                            (((1,), (1,)), ((), ())),
                            preferred_element_type=jnp.float32)

    iota = jax.lax.broadcasted_iota(jnp.int32, (tq, m), 1)
    val_cols = []
    idx_cols = []
    for _ in range(_TOPK):
        mk = jnp.max(s, axis=1, keepdims=True)
        eq = s == mk
        amin = jnp.min(jnp.where(eq, iota, m), axis=1, keepdims=True)
        s = jnp.where(eq, _NEG, s)
        val_cols.append(mk)
        idx_cols.append(amin)
    pad = 128 - _TOPK
    vals_ref[...] = jnp.concatenate(
        val_cols + [jnp.zeros((tq, pad), jnp.float32)], axis=1)
    idx_ref[...] = jnp.concatenate(
        idx_cols + [jnp.zeros((tq, pad), jnp.int32)], axis=1)


def _lane_perm(v, idx):
    dn = jax.lax.GatherDimensionNumbers(
        offset_dims=(), collapsed_slice_dims=(0,), start_index_map=(0,))
    return jax.lax.gather(v, idx[:, None], dn, slice_sizes=(1,),
                          mode=jax.lax.GatherScatterMode.PROMISE_IN_BOUNDS)


def _all_max16(v):
    lanes = jax.lax.iota(jnp.int32, 16)
    for sh in (8, 4, 2, 1):
        v = jnp.maximum(v, _lane_perm(v, lanes ^ sh))
    return v


def _all_sum16(v):
    lanes = jax.lax.iota(jnp.int32, 16)
    for sh in (8, 4, 2, 1):
        v = v + _lane_perm(v, lanes ^ sh)
    return v


def _make_sc_combine(total, c, m, n_workers):
    q_per_w = total // n_workers
    qc = 4
    n_sub = q_per_w // qc
    n_pairs = n_sub // 2
    rows_per_sub = qc * _TOPK

    mesh = plsc.VectorSubcoreMesh(core_axis_name="c", subcore_axis_name="s")

    bank_types = [
        pltpu.VMEM((qc, 128), jnp.int32),
        pltpu.VMEM((qc, 128), jnp.float32),
        pltpu.VMEM((rows_per_sub,), jnp.int32),
        pltpu.VMEM((rows_per_sub, c), jnp.float32),
        pltpu.VMEM((qc, c), jnp.float32),
    ]

    @functools.partial(
        pl.kernel, mesh=mesh,
        out_type=jax.ShapeDtypeStruct((total, c), jnp.float32),
        scratch_types=bank_types + bank_types + [pltpu.SemaphoreType.DMA((2,))],
    )
    def sc_combine(idx_hbm, vals_hbm, ltm_hbm, mem_hbm, out_hbm,
                   it0, vt0, if0, rv0, ov0, it1, vt1, if1, rv1, ov1, sem):
        wid = lax.axis_index("s") * 2 + lax.axis_index("c")
        banks = ((it0, vt0, if0, rv0, ov0, 0), (it1, vt1, if1, rv1, ov1, 1))

        def prep(bank, cidx):
            it, vt, iff, rv, ov, sidx = bank
            qb = wid * q_per_w + cidx * qc
            pltpu.sync_copy(idx_hbm.at[pl.ds(qb, qc)], it)
            pltpu.sync_copy(vals_hbm.at[pl.ds(qb, qc)], vt)
            pltpu.sync_copy(mem_hbm.at[pl.ds(qb, qc)], ov)
            for i in range(qc):
                iff[pl.ds(i * 16, 16)] = it[i, pl.ds(0, 16)]
            pltpu.async_copy(ltm_hbm.at[iff], rv, sem.at[sidx])

        def compute(bank, cidx):
            it, vt, iff, rv, ov, sidx = bank
            pltpu.make_async_copy(ltm_hbm.at[iff], rv, sem.at[sidx]).wait()
            for i in range(qc):
                v16 = vt[i, pl.ds(0, 16)]
                m1 = _all_max16(v16)
                e = jnp.exp(v16 - m1)
                w = e * (1.0 / _all_sum16(e))
                splats = [
                    _lane_perm(w, jnp.full((16,), k, jnp.int32))
                    for k in range(16)
                ]

                def dim_body(j, _):
                    sl = pl.ds(j * 16, 16)
                    acc = splats[0] * rv[i * 16, sl]
                    for k in range(1, 16):
                        acc = acc + splats[k] * rv[i * 16 + k, sl]
                    ov[i, sl] = ov[i, sl] + acc
                    return 0

                lax.fori_loop(0, c // 16, dim_body, 0)
            qb = wid * q_per_w + cidx * qc
            pltpu.sync_copy(ov, out_hbm.at[pl.ds(qb, qc)])

        prep(banks[0], 0)

        def pair_body(t, carry):
            c0 = 2 * t
            c1 = c0 + 1
            c2 = jnp.minimum(c0 + 2, n_sub - 1)
            prep(banks[1], c1)
            compute(banks[0], c0)
            prep(banks[0], c2)
            compute(banks[1], c1)
            return carry

        lax.fori_loop(0, n_pairs, pair_body, 0)
        pltpu.make_async_copy(ltm_hbm.at[if0], rv0, sem.at[0]).wait()

    return sc_combine


@jax.jit
def kernel(x, ltm_buffer, Wt1, bt1, Wt2, bt2):
    b, t, c = x.shape
    m = ltm_buffer.shape[0]
    total = b * t
    tq = 128
    n_tiles = total // tq

    xf = x.reshape(total, c)
    b1r = bt1.reshape(1, -1)
    b2r = bt2.reshape(1, -1)
    w1b = Wt1.astype(jnp.bfloat16)
    w2b = Wt2.astype(jnp.bfloat16)

    mn_bf = pl.pallas_call(
        _prep_body,
        grid=(8,),
        in_specs=[pl.BlockSpec((m // 8, c), lambda i: (i, 0))],
        out_specs=pl.BlockSpec((m // 8, c), lambda i: (i, 0)),
        out_shape=jax.ShapeDtypeStruct((m, c), jnp.bfloat16),
    )(ltm_buffer)

    mem, vals, idxs = pl.pallas_call(
        _main_body,
        grid=(n_tiles,),
        in_specs=[
            pl.BlockSpec((tq, c), lambda i: (i, 0)),
            pl.BlockSpec(memory_space=pl.ANY),
            pl.BlockSpec(memory_space=pl.ANY),
            pl.BlockSpec(memory_space=pl.ANY),
            pl.BlockSpec((1, b1r.shape[1]), lambda i: (0, 0)),
            pl.BlockSpec((1, b2r.shape[1]), lambda i: (0, 0)),
        ],
        out_specs=[pl.BlockSpec((tq, c), lambda i: (i, 0)),
                   pl.BlockSpec((tq, 128), lambda i: (i, 0)),
                   pl.BlockSpec((tq, 128), lambda i: (i, 0))],
        out_shape=[jax.ShapeDtypeStruct((total, c), jnp.float32),
                   jax.ShapeDtypeStruct((total, 128), jnp.float32),
                   jax.ShapeDtypeStruct((total, 128), jnp.int32)],
        scratch_shapes=[
            pltpu.VMEM((m, c), jnp.bfloat16),
            pltpu.VMEM((c, 2 * c), jnp.bfloat16),
            pltpu.VMEM((2 * c, c), jnp.bfloat16),
            pltpu.SemaphoreType.DMA((3,)),
        ],
        compiler_params=pltpu.CompilerParams(
            dimension_semantics=("arbitrary",),
        ),
    )(xf, mn_bf, w1b, w2b, b1r, b2r)

    sc_combine = _make_sc_combine(total, c, m, 32)
    out = sc_combine(idxs, vals, ltm_buffer, mem)
    return out.reshape(b, t, c)


# speedup vs baseline: 1.0972x; 1.0972x over previous
import functools

import jax
import jax.numpy as jnp
from jax import lax
from jax.experimental import pallas as pl
from jax.experimental.pallas import tpu as pltpu
from jax.experimental.pallas import tpu_sc as plsc

_TOPK = 16
_NEG = -1e30


def _prep_body(ltm_ref, mn_ref):
    v = ltm_ref[...]
    inv = jax.lax.rsqrt(jnp.maximum(jnp.sum(v * v, axis=1, keepdims=True), 1e-24))
    mn_ref[...] = (v * inv).astype(jnp.bfloat16)


def _main_body(x_ref, mn_hbm, w1_hbm, w2_hbm, b1_ref, b2_ref,
               mem_ref, vals_ref, idx_ref, mn_v, w1_v, w2_v, sem):
    @pl.when(pl.program_id(0) == 0)
    def _stage():
        cps = [
            pltpu.make_async_copy(mn_hbm, mn_v, sem.at[0]),
            pltpu.make_async_copy(w1_hbm, w1_v, sem.at[1]),
            pltpu.make_async_copy(w2_hbm, w2_v, sem.at[2]),
        ]
        for c in cps:
            c.start()
        for c in cps:
            c.wait()

    x = x_ref[...]
    tq = x.shape[0]
    m = mn_v.shape[0]

    h = jnp.dot(x.astype(jnp.bfloat16), w1_v[...],
                preferred_element_type=jnp.float32) + b1_ref[...]
    h = h * 0.5 * (1.0 + jax.lax.erf(h * 0.7071067811865476))
    mem = jnp.dot(h.astype(jnp.bfloat16), w2_v[...],
                  preferred_element_type=jnp.float32) + b2_ref[...]
    mem_ref[...] = mem

    qn = x * jax.lax.rsqrt(jnp.maximum(jnp.sum(x * x, axis=1, keepdims=True), 1e-24))
    s = jax.lax.dot_general(qn.astype(jnp.bfloat16), mn_v[...],
                            (((1,), (1,)), ((), ())),
                            preferred_element_type=jnp.float32)

    iota = jax.lax.broadcasted_iota(jnp.int32, (tq, m), 1)
    val_cols = []
    idx_cols = []
    for _ in range(_TOPK):
        mk = jnp.max(s, axis=1, keepdims=True)
        eq = s == mk
        amin = jnp.min(jnp.where(eq, iota, m), axis=1, keepdims=True)
        s = jnp.where(eq, _NEG, s)
        val_cols.append(mk)
        idx_cols.append(amin)
    pad = 128 - _TOPK
    vals_ref[...] = jnp.concatenate(
        val_cols + [jnp.zeros((tq, pad), jnp.float32)], axis=1)
    idx_ref[...] = jnp.concatenate(
        idx_cols + [jnp.zeros((tq, pad), jnp.int32)], axis=1)


def _lane_perm(v, idx):
    dn = jax.lax.GatherDimensionNumbers(
        offset_dims=(), collapsed_slice_dims=(0,), start_index_map=(0,))
    return jax.lax.gather(v, idx[:, None], dn, slice_sizes=(1,),
                          mode=jax.lax.GatherScatterMode.PROMISE_IN_BOUNDS)


def _all_max16(v):
    lanes = jax.lax.iota(jnp.int32, 16)
    for sh in (8, 4, 2, 1):
        v = jnp.maximum(v, _lane_perm(v, lanes ^ sh))
    return v


def _all_sum16(v):
    lanes = jax.lax.iota(jnp.int32, 16)
    for sh in (8, 4, 2, 1):
        v = v + _lane_perm(v, lanes ^ sh)
    return v


def _make_sc_combine(total, c, m, n_workers):
    q_per_w = total // n_workers
    qc = 8
    n_sub = q_per_w // qc
    rows_per_sub = qc * _TOPK

    mesh = plsc.VectorSubcoreMesh(core_axis_name="c", subcore_axis_name="s")

    @functools.partial(
        pl.kernel, mesh=mesh,
        out_type=jax.ShapeDtypeStruct((total, c), jnp.float32),
        scratch_types=[
            pltpu.VMEM((qc, 128), jnp.int32),
            pltpu.VMEM((qc, 128), jnp.float32),
            pltpu.VMEM((rows_per_sub,), jnp.int32),
            pltpu.VMEM((rows_per_sub, c), jnp.float32),
            pltpu.VMEM((qc, c), jnp.float32),
            pltpu.SemaphoreType.DMA,
        ],
    )
    def sc_combine(idx_hbm, vals_hbm, ltm_hbm, mem_hbm, out_hbm,
                   idx_t, vals_t, idx_f, rows_v, out_v, sem):
        wid = lax.axis_index("s") * 2 + lax.axis_index("c")

        def sub_body(cidx, carry):
            qb = wid * q_per_w + cidx * qc
            pltpu.sync_copy(idx_hbm.at[pl.ds(qb, qc)], idx_t)
            pltpu.sync_copy(vals_hbm.at[pl.ds(qb, qc)], vals_t)
            pltpu.sync_copy(mem_hbm.at[pl.ds(qb, qc)], out_v)
            for i in range(qc):
                idx_f[pl.ds(i * 16, 16)] = idx_t[i, pl.ds(0, 16)]
            pltpu.async_copy(ltm_hbm.at[idx_f], rows_v, sem).wait()
            for i in range(qc):
                v16 = vals_t[i, pl.ds(0, 16)]
                m1 = _all_max16(v16)
                e = jnp.exp(v16 - m1)
                w = e * (1.0 / _all_sum16(e))
                splats = [
                    _lane_perm(w, jnp.full((16,), k, jnp.int32))
                    for k in range(16)
                ]

                def dim_body(j, _):
                    sl = pl.ds(j * 16, 16)
                    acc = splats[0] * rows_v[i * 16, sl]
                    for k in range(1, 16):
                        acc = acc + splats[k] * rows_v[i * 16 + k, sl]
                    out_v[i, sl] = out_v[i, sl] + acc
                    return 0

                lax.fori_loop(0, c // 16, dim_body, 0)
            pltpu.sync_copy(out_v, out_hbm.at[pl.ds(qb, qc)])
            return carry

        lax.fori_loop(0, n_sub, sub_body, 0)

    return sc_combine


@jax.jit
def kernel(x, ltm_buffer, Wt1, bt1, Wt2, bt2):
    b, t, c = x.shape
    m = ltm_buffer.shape[0]
    total = b * t
    tq = 128
    n_tiles = total // tq

    xf = x.reshape(total, c)
    b1r = bt1.reshape(1, -1)
    b2r = bt2.reshape(1, -1)
    w1b = Wt1.astype(jnp.bfloat16)
    w2b = Wt2.astype(jnp.bfloat16)

    mn_bf = pl.pallas_call(
        _prep_body,
        grid=(8,),
        in_specs=[pl.BlockSpec((m // 8, c), lambda i: (i, 0))],
        out_specs=pl.BlockSpec((m // 8, c), lambda i: (i, 0)),
        out_shape=jax.ShapeDtypeStruct((m, c), jnp.bfloat16),
    )(ltm_buffer)

    half = total // 2
    sc_combine = _make_sc_combine(half, c, m, 32)

    def tc_half(xh):
        return pl.pallas_call(
            _main_body,
            grid=(half // tq,),
            in_specs=[
                pl.BlockSpec((tq, c), lambda i: (i, 0)),
                pl.BlockSpec(memory_space=pl.ANY),
                pl.BlockSpec(memory_space=pl.ANY),
                pl.BlockSpec(memory_space=pl.ANY),
                pl.BlockSpec((1, b1r.shape[1]), lambda i: (0, 0)),
                pl.BlockSpec((1, b2r.shape[1]), lambda i: (0, 0)),
            ],
            out_specs=[pl.BlockSpec((tq, c), lambda i: (i, 0)),
                       pl.BlockSpec((tq, 128), lambda i: (i, 0)),
                       pl.BlockSpec((tq, 128), lambda i: (i, 0))],
            out_shape=[jax.ShapeDtypeStruct((half, c), jnp.float32),
                       jax.ShapeDtypeStruct((half, 128), jnp.float32),
                       jax.ShapeDtypeStruct((half, 128), jnp.int32)],
            scratch_shapes=[
                pltpu.VMEM((m, c), jnp.bfloat16),
                pltpu.VMEM((c, 2 * c), jnp.bfloat16),
                pltpu.VMEM((2 * c, c), jnp.bfloat16),
                pltpu.SemaphoreType.DMA((3,)),
            ],
            compiler_params=pltpu.CompilerParams(
                dimension_semantics=("arbitrary",),
            ),
        )(xh, mn_bf, w1b, w2b, b1r, b2r)

    outs = []
    for hh in range(2):
        mem_h, vals_h, idx_h = tc_half(
            jax.lax.slice_in_dim(xf, hh * half, (hh + 1) * half, axis=0))
        outs.append(sc_combine(idx_h, vals_h, ltm_buffer, mem_h))
    out = jnp.concatenate(outs, axis=0)
    return out.reshape(b, t, c)
